# baseline (speedup 1.0000x reference)
import jax
import jax.numpy as jnp
from jax import lax
from jax.experimental import pallas as pl
from jax.experimental.pallas import tpu as pltpu
from jax.experimental.pallas import tpu_sc as plsc

N = 100000
D = 128
G = 256
EPS = 1e-8

NC = 2
NS = 16
NW = NC * NS
VPR = D // 16
CROWS = 125
NCHUNK = 25
BROWS = N // CROWS
GSEG_PAD = 264


def _segsum_body(x1_hbm, b_hbm, x2_hbm, p_out, xbuf, idbuf, zbuf, accs, sem):
    sid = lax.axis_index("s")
    wid = sid * NC + lax.axis_index("c")
    zerov = jnp.zeros((16,), jnp.float32)

    @pl.loop(0, GSEG_PAD)
    def _zero(r):
        for j in range(VPR):
            zbuf[r, pl.ds(j * 16, 16)] = zerov

    def _start(a, c):
        row0 = (wid * NCHUNK + c) * CROWS
        dst = xbuf.at[pl.ds((c % 2) * CROWS, CROWS)]

        @pl.when(a == 0)
        def _():
            pltpu.async_copy(x1_hbm.at[pl.ds(row0, CROWS)], dst, sem)

        @pl.when(a == 1)
        def _():
            pltpu.async_copy(x2_hbm.at[pl.ds(row0, CROWS)], dst, sem)

    def _wait_one():
        pltpu.make_async_copy(
            x1_hbm.at[pl.ds(0, CROWS)], xbuf.at[pl.ds(0, CROWS)], sem
        ).wait()

    @pl.loop(0, 2)
    def _per_array(a):
        _start(a, 0)
        pltpu.sync_copy(
            b_hbm.at[pl.ds(a * BROWS + wid * NCHUNK, NCHUNK)], idbuf
        )

        @pl.when(sid == 0)
        def _():
            pltpu.sync_copy(zbuf, accs)

        plsc.subcore_barrier()

        @pl.loop(0, NCHUNK)
        def _chunk(c):
            _wait_one()

            @pl.when(c + 1 < NCHUNK)
            def _():
                _start(a, c + 1)

            pltpu.sync_copy(
                xbuf.at[pl.ds((c % 2) * CROWS, CROWS)],
                accs.at[idbuf.at[c]],
                add=True,
            )

        plsc.subcore_barrier()

        @pl.when(sid == 0)
        def _():
            pltpu.sync_copy(accs, p_out.at[a, lax.axis_index("c")])

        plsc.subcore_barrier()


@jax.jit
def _segsum(x1, b2d, x2):
    mesh = plsc.VectorSubcoreMesh(
        core_axis_name="c", subcore_axis_name="s", num_cores=NC
    )
    return pl.kernel(
        _segsum_body,
        out_type=jax.ShapeDtypeStruct((2, NC, GSEG_PAD, D), jnp.float32),
        mesh=mesh,
        compiler_params=pltpu.CompilerParams(
            use_tc_tiling_on_sc=False, needs_layout_passes=False
        ),
        scratch_types=[
            pltpu.VMEM((2 * CROWS, D), jnp.float32),
            pltpu.VMEM((NCHUNK, CROWS), jnp.int32),
            pltpu.VMEM((GSEG_PAD, D), jnp.float32),
            pltpu.VMEM_SHARED((GSEG_PAD, D), jnp.float32),
            pltpu.SemaphoreType.DMA,
        ],
    )(x1, b2d, x2)


def _finish_body(p_ref, w_ref, out_ref):
    w = w_ref[...]

    def embed(a):
        s = jnp.sum(p_ref[a], axis=0)[:G]
        return jnp.dot(s, w, preferred_element_type=jnp.float32)

    e1 = embed(0)
    e2 = embed(1)
    n1 = jnp.maximum(jnp.sqrt(jnp.sum(e1 * e1, axis=-1)), EPS)
    n2 = jnp.maximum(jnp.sqrt(jnp.sum(e2 * e2, axis=-1)), EPS)
    out_ref[...] = jnp.sum(e1 * e2, axis=-1) / (n1 * n2)


@jax.jit
def _finish(part, w):
    return pl.pallas_call(
        _finish_body,
        out_shape=jax.ShapeDtypeStruct((G,), jnp.float32),
    )(part, w)


def kernel(x1, batch1, x2, batch2, W):
    b2d = jnp.concatenate(
        [batch1.astype(jnp.int32), batch2.astype(jnp.int32)]
    ).reshape(2 * BROWS, CROWS)
    part = _segsum(x1, b2d, x2)
    return _finish(part, W)


# speedup vs baseline: 1.4078x; 1.4078x over previous
import jax
import jax.numpy as jnp
from jax import lax
from jax.experimental import pallas as pl
from jax.experimental.pallas import tpu as pltpu
from jax.experimental.pallas import tpu_sc as plsc

N = 100000
D = 128
G = 256
EPS = 1e-8

NC = 2
NS = 16
NW = NC * NS
VPR = D // 16
NGRP = N // 16
GPW = 200
CHUNK_G = 10
CROWS = CHUNK_G * 16
GSEG_PAD = 264


def _extract(vec, i):
    return lax.squeeze(lax.slice_in_dim(vec, i, i + 1), (0,))


def _segsum_body(x1_hbm, b1_hbm, x2_hbm, b2_hbm, p_out,
                 xbuf, idbuf, acc, sem):
    wid = lax.axis_index("s") * NC + lax.axis_index("c")
    g0 = wid * GPW
    nchunk = jnp.minimum(GPW // CHUNK_G, (NGRP - g0) // CHUNK_G)
    last = g0 + GPW > NGRP
    zerov = jnp.zeros((16,), jnp.float32)

    def _start(a, c):
        row0 = (g0 + c * CHUNK_G) * 16
        dst = xbuf.at[pl.ds((c % 3) * CROWS, CROWS)]

        @pl.when(a == 0)
        def _():
            pltpu.async_copy(x1_hbm.at[pl.ds(row0, CROWS)], dst, sem)

        @pl.when(a == 1)
        def _():
            pltpu.async_copy(x2_hbm.at[pl.ds(row0, CROWS)], dst, sem)

    def _wait_one():
        pltpu.make_async_copy(
            x1_hbm.at[pl.ds(0, CROWS)], xbuf.at[pl.ds(0, CROWS)], sem
        ).wait()

    @pl.loop(0, 2)
    def _per_array(a):
        _start(a, 0)
        _start(a, 1)

        i0 = g0 * 16
        for aa, b_hbm in ((0, b1_hbm), (1, b2_hbm)):
            @pl.when((a == aa) & jnp.logical_not(last))
            def _():
                pltpu.sync_copy(b_hbm.at[pl.ds(i0, GPW * 16)], idbuf)

            @pl.when((a == aa) & last)
            def _():
                pltpu.sync_copy(b_hbm.at[pl.ds(i0, 800)],
                                idbuf.at[pl.ds(0, 800)])

        @pl.loop(0, GSEG_PAD)
        def _zero(r):
            for j in range(VPR):
                acc[r, pl.ds(j * 16, 16)] = zerov

        cur0 = _extract(idbuf[pl.ds(0, 16)], 0)
        carry0 = (cur0,) + (zerov,) * VPR

        def _chunk(c, carry):
            _wait_one()

            @pl.when(c + 2 < nchunk)
            def _():
                _start(a, c + 2)

            boff = (c % 3) * CROWS

            def _group(g, carry):
                idv = idbuf[pl.ds((c * CHUNK_G + g) * 16, 16)]
                base = boff + g * 16
                cur = carry[0]
                same = jnp.all(idv == cur)

                def _fast(cur, *regs):
                    new = []
                    for j in range(VPR):
                        s = regs[j]
                        for i in range(16):
                            s = s + xbuf[base + i, pl.ds(j * 16, 16)]
                        new.append(s)
                    return (cur, *new)

                def _slow(cur, *regs):
                    regs = list(regs)
                    for i in range(16):
                        b = _extract(idv, i)

                        def _flush(cur, *regs):
                            for j in range(VPR):
                                plsc.addupdate(
                                    acc.at[cur, pl.ds(j * 16, 16)], regs[j]
                                )
                            return (b,) + (zerov,) * VPR

                        def _keep(cur, *regs):
                            return (cur, *regs)

                        cur, *regs = lax.cond(b != cur, _flush, _keep,
                                              cur, *regs)
                        for j in range(VPR):
                            regs[j] = regs[j] + xbuf[base + i,
                                                     pl.ds(j * 16, 16)]
                    return (cur, *regs)

                return lax.cond(same, _fast, _slow, *carry)

            return lax.fori_loop(0, CHUNK_G, _group, carry)

        cur, *regs = lax.fori_loop(0, nchunk, _chunk, carry0)
        for j in range(VPR):
            plsc.addupdate(acc.at[cur, pl.ds(j * 16, 16)], regs[j])

        pltpu.sync_copy(acc, p_out.at[a, wid])


@jax.jit
def _segsum(x1, b1, x2, b2):
    mesh = plsc.VectorSubcoreMesh(
        core_axis_name="c", subcore_axis_name="s", num_cores=NC
    )
    return pl.kernel(
        _segsum_body,
        out_type=jax.ShapeDtypeStruct((2, NW, GSEG_PAD, D), jnp.float32),
        mesh=mesh,
        compiler_params=pltpu.CompilerParams(
            use_tc_tiling_on_sc=False, needs_layout_passes=False
        ),
        scratch_types=[
            pltpu.VMEM((3 * CROWS, D), jnp.float32),
            pltpu.VMEM((GPW * 16,), jnp.int32),
            pltpu.VMEM((GSEG_PAD, D), jnp.float32),
            pltpu.SemaphoreType.DMA,
        ],
    )(x1, b1, x2, b2)


def _finish_body(p_ref, w_ref, out_ref):
    w = w_ref[...]

    def embed(a):
        s = jnp.sum(p_ref[a], axis=0)[:G]
        return jnp.dot(s, w, preferred_element_type=jnp.float32)

    e1 = embed(0)
    e2 = embed(1)
    n1 = jnp.maximum(jnp.sqrt(jnp.sum(e1 * e1, axis=-1)), EPS)
    n2 = jnp.maximum(jnp.sqrt(jnp.sum(e2 * e2, axis=-1)), EPS)
    out_ref[...] = jnp.sum(e1 * e2, axis=-1) / (n1 * n2)


@jax.jit
def _finish(part, w):
    return pl.pallas_call(
        _finish_body,
        out_shape=jax.ShapeDtypeStruct((G,), jnp.float32),
    )(part, w)


def kernel(x1, batch1, x2, batch2, W):
    part = _segsum(x1, batch1.astype(jnp.int32), x2,
                   batch2.astype(jnp.int32))
    return _finish(part, W)


# speedup vs baseline: 1.4086x; 1.0006x over previous
import jax
import jax.numpy as jnp
from jax import lax
from jax.experimental import pallas as pl
from jax.experimental.pallas import tpu as pltpu
from jax.experimental.pallas import tpu_sc as plsc

N = 100000
D = 128
G = 256
EPS = 1e-8

NC = 2
NS = 16
NW = NC * NS
VPR = D // 16
NGRP = N // 16
GPW = 200
CHUNK_G = 10
CROWS = CHUNK_G * 16
GSEG_PAD = 264


def _extract(vec, i):
    return lax.squeeze(lax.slice_in_dim(vec, i, i + 1), (0,))


def _segsum_body(x1_hbm, b1_hbm, x2_hbm, b2_hbm, p_out,
                 xbuf, idbuf, acc, sem):
    wid = lax.axis_index("s") * NC + lax.axis_index("c")
    g0 = wid * GPW
    nchunk = jnp.minimum(GPW // CHUNK_G, (NGRP - g0) // CHUNK_G)
    last = g0 + GPW > NGRP
    zerov = jnp.zeros((16,), jnp.float32)

    def _start(a, c):
        row0 = (g0 + c * CHUNK_G) * 16
        dst = xbuf.at[pl.ds((c % 4) * CROWS, CROWS)]

        @pl.when(a == 0)
        def _():
            pltpu.async_copy(x1_hbm.at[pl.ds(row0, CROWS)], dst, sem)

        @pl.when(a == 1)
        def _():
            pltpu.async_copy(x2_hbm.at[pl.ds(row0, CROWS)], dst, sem)

    def _wait_one():
        pltpu.make_async_copy(
            x1_hbm.at[pl.ds(0, CROWS)], xbuf.at[pl.ds(0, CROWS)], sem
        ).wait()

    @pl.loop(0, 2)
    def _per_array(a):
        _start(a, 0)
        _start(a, 1)
        _start(a, 2)

        i0 = g0 * 16
        for aa, b_hbm in ((0, b1_hbm), (1, b2_hbm)):
            @pl.when((a == aa) & jnp.logical_not(last))
            def _():
                pltpu.sync_copy(b_hbm.at[pl.ds(i0, GPW * 16)], idbuf)

            @pl.when((a == aa) & last)
            def _():
                pltpu.sync_copy(b_hbm.at[pl.ds(i0, 800)],
                                idbuf.at[pl.ds(0, 800)])

        @pl.loop(0, GSEG_PAD)
        def _zero(r):
            for j in range(VPR):
                acc[r, pl.ds(j * 16, 16)] = zerov

        cur0 = _extract(idbuf[pl.ds(0, 16)], 0)
        carry0 = (cur0,) + (zerov,) * VPR

        def _chunk(c, carry):
            _wait_one()

            @pl.when(c + 3 < nchunk)
            def _():
                _start(a, c + 3)

            boff = (c % 4) * CROWS

            def _group(g, carry):
                idv = idbuf[pl.ds((c * CHUNK_G + g) * 16, 16)]
                base = boff + g * 16
                cur = carry[0]
                same = jnp.all(idv == cur)

                def _fast(cur, *regs):
                    new = []
                    for j in range(VPR):
                        s = regs[j]
                        for i in range(16):
                            s = s + xbuf[base + i, pl.ds(j * 16, 16)]
                        new.append(s)
                    return (cur, *new)

                def _slow(cur, *regs):
                    regs = list(regs)
                    for i in range(16):
                        b = _extract(idv, i)

                        def _flush(cur, *regs):
                            for j in range(VPR):
                                plsc.addupdate(
                                    acc.at[cur, pl.ds(j * 16, 16)], regs[j]
                                )
                            return (b,) + (zerov,) * VPR

                        def _keep(cur, *regs):
                            return (cur, *regs)

                        cur, *regs = lax.cond(b != cur, _flush, _keep,
                                              cur, *regs)
                        for j in range(VPR):
                            regs[j] = regs[j] + xbuf[base + i,
                                                     pl.ds(j * 16, 16)]
                    return (cur, *regs)

                return lax.cond(same, _fast, _slow, *carry)

            return lax.fori_loop(0, CHUNK_G, _group, carry)

        cur, *regs = lax.fori_loop(0, nchunk, _chunk, carry0)
        for j in range(VPR):
            plsc.addupdate(acc.at[cur, pl.ds(j * 16, 16)], regs[j])

        pltpu.sync_copy(acc, p_out.at[a, wid])


@jax.jit
def _segsum(x1, b1, x2, b2):
    mesh = plsc.VectorSubcoreMesh(
        core_axis_name="c", subcore_axis_name="s", num_cores=NC
    )
    return pl.kernel(
        _segsum_body,
        out_type=jax.ShapeDtypeStruct((2, NW, GSEG_PAD, D), jnp.float32),
        mesh=mesh,
        compiler_params=pltpu.CompilerParams(
            use_tc_tiling_on_sc=False, needs_layout_passes=False
        ),
        scratch_types=[
            pltpu.VMEM((4 * CROWS, D), jnp.float32),
            pltpu.VMEM((GPW * 16,), jnp.int32),
            pltpu.VMEM((GSEG_PAD, D), jnp.float32),
            pltpu.SemaphoreType.DMA,
        ],
    )(x1, b1, x2, b2)


def _finish_body(p_ref, w_ref, out_ref):
    w = w_ref[...]

    def embed(a):
        s = jnp.sum(p_ref[a], axis=0)[:G]
        return jnp.dot(s, w, preferred_element_type=jnp.float32)

    e1 = embed(0)
    e2 = embed(1)
    n1 = jnp.maximum(jnp.sqrt(jnp.sum(e1 * e1, axis=-1)), EPS)
    n2 = jnp.maximum(jnp.sqrt(jnp.sum(e2 * e2, axis=-1)), EPS)
    out_ref[...] = jnp.sum(e1 * e2, axis=-1) / (n1 * n2)


@jax.jit
def _finish(part, w):
    return pl.pallas_call(
        _finish_body,
        out_shape=jax.ShapeDtypeStruct((G,), jnp.float32),
    )(part, w)


def kernel(x1, batch1, x2, batch2, W):
    part = _segsum(x1, batch1.astype(jnp.int32), x2,
                   batch2.astype(jnp.int32))
    return _finish(part, W)
